# baseline (speedup 1.0000x reference)
import functools

import jax
import jax.numpy as jnp
from jax.experimental import pallas as pl
from jax.experimental.pallas import tpu as pltpu

_M = 32


def _bp_kernel(llr_ref, w_ref, s_ref, t_ref, chat_ref, loss_ref, *, num_iter,
               inv_denom):
    f32 = jnp.float32
    bf16 = jnp.bfloat16
    S = s_ref[...]
    T = t_ref[...]

    def dot_seg(a, b):
        return jax.lax.dot_general(a, b, (((1,), (0,)), ((), ())),
                                   preferred_element_type=f32)

    def dot_gat(a, b):
        return jax.lax.dot_general(a, b, (((1,), (1,)), ((), ())),
                                   preferred_element_type=f32)

    def split2(a):
        h = a.astype(bf16)
        return h, (a - h.astype(f32)).astype(bf16)

    def seg(a, b):
        h, l = split2(a)
        n = a.shape[0]
        g = dot_seg(jnp.concatenate([h, l], axis=0), b)
        return g[:n] + g[n:]

    def gat(a, b):
        h, l = split2(a)
        n = a.shape[0]
        g = dot_gat(jnp.concatenate([h, l], axis=0), b)
        return g[:n] + g[n:]

    deg = jnp.sum(S.astype(f32), axis=0, keepdims=True)

    def chain(llr_blk):
        llr_c = jnp.clip(llr_blk, -20.0, 20.0)
        msg_vn = gat(llr_c, T)
        loss_acc = jnp.zeros((), f32)
        total = llr_c
        for i in range(num_iter):
            neg = msg_vn < 0.0
            t = jnp.tanh(jnp.abs(msg_vn) * 0.5)
            log_mag = jnp.log(jnp.maximum(t, 1e-12))
            flip = jnp.where(neg, 1.0, -1.0)
            sum_log = seg(log_mag, S)
            neg_cnt = (dot_seg(flip.astype(bf16), S) + deg) * 0.5
            sign_total = 1.0 - 2.0 * jnp.mod(neg_cnt, 2.0)
            bh = sum_log.shape[0]
            sh, sl = split2(sum_log)
            g3 = dot_gat(jnp.concatenate(
                [sh, sl, sign_total.astype(bf16)], axis=0), S)
            log_excl = (g3[:bh] + g3[bh:2 * bh]) - log_mag
            flip_excl = g3[2 * bh:] * flip
            y = jnp.maximum(log_excl * -0.5, 5.00000025e-08)
            msg_cn = flip_excl * jnp.log(jnp.tanh(y)) * w_ref[i][None, :]
            total = llr_c + seg(msg_cn, T)
            msg_vn = gat(total, T) - msg_cn
            x = -total
            loss_acc = loss_acc + jnp.sum(
                jnp.maximum(x, 0.0) + jnp.log1p(jnp.exp(-jnp.abs(x))))
        return total, loss_acc

    bb = llr_ref.shape[0]
    hb = bb // 2
    total0, loss0 = chain(llr_ref[:hb, :])
    total1, loss1 = chain(llr_ref[hb:, :])
    chat_ref[:hb, :] = total0
    chat_ref[hb:, :] = total1
    loss_ref[...] = ((loss0 + loss1) * inv_denom).reshape(1, 1, 1)


def kernel(llr, W, edge_cn, edge_vn):
    B, N = llr.shape
    num_iter, E = W.shape
    f32 = jnp.float32
    S = (edge_cn[:, None] == jnp.arange(_M, dtype=edge_cn.dtype)[None, :]
         ).astype(jnp.bfloat16)
    T = (edge_vn[:, None] == jnp.arange(N, dtype=edge_vn.dtype)[None, :]
         ).astype(jnp.bfloat16)

    bb = 4096
    grid = B // bb
    c_hat, loss = pl.pallas_call(
        functools.partial(_bp_kernel, num_iter=num_iter,
                          inv_denom=1.0 / (float(B * N) * float(num_iter))),
        grid=(grid,),
        in_specs=[
            pl.BlockSpec((bb, N), lambda i: (i, 0)),
            pl.BlockSpec((num_iter, E), lambda i: (0, 0)),
            pl.BlockSpec((E, _M), lambda i: (0, 0)),
            pl.BlockSpec((E, N), lambda i: (0, 0)),
        ],
        out_specs=[
            pl.BlockSpec((bb, N), lambda i: (i, 0)),
            pl.BlockSpec((1, 1, 1), lambda i: (i, 0, 0)),
        ],
        out_shape=[
            jax.ShapeDtypeStruct((B, N), f32),
            jax.ShapeDtypeStruct((grid, 1, 1), f32),
        ],
        compiler_params=pltpu.CompilerParams(
            dimension_semantics=("parallel",)),
    )(llr, W, S, T)
    return jnp.zeros((B, N), llr.dtype), c_hat, jnp.sum(loss)


# speedup vs baseline: 1.0163x; 1.0163x over previous
import functools

import jax
import jax.numpy as jnp
from jax.experimental import pallas as pl
from jax.experimental.pallas import tpu as pltpu

_M = 32


def _bp_kernel(llr_ref, w_ref, s_ref, t_ref, chat_ref, loss_ref, *, num_iter,
               inv_denom):
    f32 = jnp.float32
    bf16 = jnp.bfloat16
    S = s_ref[...]
    T = t_ref[...]

    def dot_seg(a, b):
        return jax.lax.dot_general(a, b, (((1,), (0,)), ((), ())),
                                   preferred_element_type=f32)

    def dot_gat(a, b):
        return jax.lax.dot_general(a, b, (((1,), (1,)), ((), ())),
                                   preferred_element_type=f32)

    def split2(a):
        h = a.astype(bf16)
        return h, (a - h.astype(f32)).astype(bf16)

    def seg(a, b):
        h, l = split2(a)
        return dot_seg(h, b) + dot_seg(l, b)

    def gat(a, b):
        h, l = split2(a)
        n = a.shape[0]
        g = dot_gat(jnp.concatenate([h, l], axis=0), b)
        return g[:n] + g[n:]

    deg = jnp.sum(S.astype(f32), axis=0, keepdims=True)

    def chain(llr_blk):
        llr_c = jnp.clip(llr_blk, -20.0, 20.0)
        msg_vn = gat(llr_c, T)
        loss_acc = jnp.zeros((), f32)
        total = llr_c
        for i in range(num_iter):
            neg = msg_vn < 0.0
            t = jnp.tanh(jnp.abs(msg_vn) * 0.5)
            log_mag = jnp.log(jnp.maximum(t, 1e-12))
            flip = jnp.where(neg, 1.0, -1.0)
            sum_log = seg(log_mag, S)
            neg_cnt = (dot_seg(flip.astype(bf16), S) + deg) * 0.5
            sign_total = 1.0 - 2.0 * jnp.mod(neg_cnt, 2.0)
            bh = sum_log.shape[0]
            sh, sl = split2(sum_log)
            g3 = dot_gat(jnp.concatenate(
                [sh, sl, sign_total.astype(bf16)], axis=0), S)
            log_excl = (g3[:bh] + g3[bh:2 * bh]) - log_mag
            flip_excl = g3[2 * bh:] * flip
            y = jnp.maximum(log_excl * -0.5, 5.00000025e-08)
            msg_cn = flip_excl * jnp.log(jnp.tanh(y)) * w_ref[i][None, :]
            total = llr_c + seg(msg_cn, T)
            msg_vn = gat(total, T) - msg_cn
            x = -total
            loss_acc = loss_acc + jnp.sum(
                jnp.maximum(x, 0.0) + jnp.log1p(jnp.exp(-jnp.abs(x))))
        return total, loss_acc

    bb = llr_ref.shape[0]
    hb = bb // 2
    total0, loss0 = chain(llr_ref[:hb, :])
    total1, loss1 = chain(llr_ref[hb:, :])
    chat_ref[:hb, :] = total0
    chat_ref[hb:, :] = total1
    loss_ref[...] = ((loss0 + loss1) * inv_denom).reshape(1, 1, 1)


def kernel(llr, W, edge_cn, edge_vn):
    B, N = llr.shape
    num_iter, E = W.shape
    f32 = jnp.float32
    S = (edge_cn[:, None] == jnp.arange(_M, dtype=edge_cn.dtype)[None, :]
         ).astype(jnp.bfloat16)
    T = (edge_vn[:, None] == jnp.arange(N, dtype=edge_vn.dtype)[None, :]
         ).astype(jnp.bfloat16)

    bb = 4096
    grid = B // bb
    c_hat, loss = pl.pallas_call(
        functools.partial(_bp_kernel, num_iter=num_iter,
                          inv_denom=1.0 / (float(B * N) * float(num_iter))),
        grid=(grid,),
        in_specs=[
            pl.BlockSpec((bb, N), lambda i: (i, 0)),
            pl.BlockSpec((num_iter, E), lambda i: (0, 0)),
            pl.BlockSpec((E, _M), lambda i: (0, 0)),
            pl.BlockSpec((E, N), lambda i: (0, 0)),
        ],
        out_specs=[
            pl.BlockSpec((bb, N), lambda i: (i, 0)),
            pl.BlockSpec((1, 1, 1), lambda i: (i, 0, 0)),
        ],
        out_shape=[
            jax.ShapeDtypeStruct((B, N), f32),
            jax.ShapeDtypeStruct((grid, 1, 1), f32),
        ],
        compiler_params=pltpu.CompilerParams(
            dimension_semantics=("parallel",)),
    )(llr, W, S, T)
    return jnp.zeros((B, N), llr.dtype), c_hat, jnp.sum(loss)
